# baseline (speedup 1.0000x reference)
import functools

import jax
import jax.numpy as jnp
from jax import lax
from jax.experimental import pallas as pl
from jax.experimental.pallas import tpu as pltpu
from jax.experimental.pallas import tpu_sc as plsc

VOCAB = 1000000
EMBED_DIM = 64
BATCH = 4096
SEQ = 200

_N = BATCH * SEQ
_NC = 2
_NS = 16
_NW = _NC * _NS
_PER_W = _N // _NW
_IDX_ROW = 128
_K = 5
_CHUNK = _K * _IDX_ROW
_STEPS = _PER_W // _CHUNK
_PAD = 2 * EMBED_DIM

assert _PER_W % _CHUNK == 0 and _STEPS % 2 == 0

_SLABS = VOCAB // 8
_AT = 24
_ANCH = _SLABS // _AT
_ATAILS = _SLABS - _ANCH * _AT
_AOUT = _AT * 4


@functools.partial(
    pl.kernel,
    mesh=plsc.VectorSubcoreMesh(core_axis_name="c", subcore_axis_name="s"),
    compiler_params=pltpu.CompilerParams(
        use_tc_tiling_on_sc=True, needs_layout_passes=False
    ),
    out_type=jax.ShapeDtypeStruct((VOCAB // 2, _PAD), jnp.float32),
    scratch_types=[
        pltpu.VMEM((2, _AT, 8, EMBED_DIM), jnp.float32),
        pltpu.VMEM((2, _AOUT, _PAD), jnp.float32),
        pltpu.SemaphoreType.DMA,
        pltpu.SemaphoreType.DMA,
    ],
)
def _depad_kernel(slab_hbm, out_hbm, in_v, pk_v, rsem, wsem):
    wid = lax.axis_index("s") * _NC + lax.axis_index("c")
    my_n = (_ANCH - wid + _NW - 1) // _NW

    def fire_read(i, b):
        c = wid + _NW * i
        pltpu.async_copy(slab_hbm.at[pl.ds(c * _AT, _AT)], in_v.at[b], rsem)

    def wait_read(i, b):
        c = wid + _NW * i
        pltpu.make_async_copy(
            slab_hbm.at[pl.ds(c * _AT, _AT)], in_v.at[b], rsem
        ).wait()

    def fire_write(i, b):
        c = wid + _NW * i
        pltpu.async_copy(pk_v.at[b], out_hbm.at[pl.ds(c * _AOUT, _AOUT)], wsem)

    def wait_write(i, b):
        c = wid + _NW * i
        pltpu.make_async_copy(
            pk_v.at[b], out_hbm.at[pl.ds(c * _AOUT, _AOUT)], wsem
        ).wait()

    def compact(b, nslabs):
        for t in range(nslabs):
            for s in range(8):
                r = 8 * t + s
                for k in range(4):
                    pk_v[b, r // 2, pl.ds((r % 2) * EMBED_DIM + 16 * k, 16)] = (
                        in_v[b, t, s, pl.ds(16 * k, 16)]
                    )

    fire_read(0, 0)

    def step(i, carry):
        b = lax.rem(i, 2)

        @pl.when(i >= 2)
        def _():
            wait_write(i - 2, b)

        @pl.when(i + 1 < my_n)
        def _():
            fire_read(i + 1, 1 - b)

        wait_read(i, b)
        compact(b, _AT)
        fire_write(i, b)
        return carry

    lax.fori_loop(0, my_n, step, 0)
    wait_write(my_n - 2, lax.rem(my_n - 2, 2))
    wait_write(my_n - 1, lax.rem(my_n - 1, 2))

    @pl.when(wid == 0)
    def _tail():
        pltpu.sync_copy(
            slab_hbm.at[pl.ds(_ANCH * _AT, _ATAILS)],
            in_v.at[0].at[pl.ds(0, _ATAILS)],
        )
        compact(0, _ATAILS)
        pltpu.sync_copy(
            pk_v.at[0].at[pl.ds(0, _ATAILS * 4)],
            out_hbm.at[pl.ds(_ANCH * _AOUT, _ATAILS * 4)],
        )


@functools.partial(
    pl.kernel,
    mesh=plsc.VectorSubcoreMesh(core_axis_name="c", subcore_axis_name="s"),
    compiler_params=pltpu.CompilerParams(use_tc_tiling_on_sc=False),
    out_type=jax.ShapeDtypeStruct((_N, _PAD), jnp.float32),
    scratch_types=[
        pltpu.VMEM((2, _K, _IDX_ROW), jnp.int32),
        pltpu.VMEM((2, _CHUNK, EMBED_DIM), jnp.float32),
        pltpu.SemaphoreType.DMA,
        pltpu.SemaphoreType.DMA,
    ],
)
def _gather_kernel(table_hbm, idx_hbm, out_hbm, idx_v, rows_v, gsem, ssem):
    wid = lax.axis_index("s") * _NC + lax.axis_index("c")
    row_base = wid * (_PER_W // _IDX_ROW)
    base = wid * _PER_W

    def fire_gathers(c, b):
        pltpu.sync_copy(idx_hbm.at[pl.ds(row_base + c * _K, _K)], idx_v.at[b])
        for j in range(_K):
            pltpu.async_copy(
                table_hbm.at[idx_v.at[b].at[j]],
                rows_v.at[b].at[pl.ds(j * _IDX_ROW, _IDX_ROW)],
                gsem,
            )

    def wait_gathers(b):
        for j in range(_K):
            pltpu.make_async_copy(
                table_hbm.at[idx_v.at[b].at[j]],
                rows_v.at[b].at[pl.ds(j * _IDX_ROW, _IDX_ROW)],
                gsem,
            ).wait()

    def fire_store(c, b):
        pltpu.async_copy(
            rows_v.at[b],
            out_hbm.at[pl.ds(base + c * _CHUNK, _CHUNK), pl.ds(0, EMBED_DIM)],
            ssem,
        )

    def wait_store(c, b):
        pltpu.make_async_copy(
            rows_v.at[b],
            out_hbm.at[pl.ds(base + c * _CHUNK, _CHUNK), pl.ds(0, EMBED_DIM)],
            ssem,
        ).wait()

    fire_gathers(0, 0)

    def step(c, carry):
        b = lax.rem(c, 2)
        nb = 1 - b

        @pl.when(c + 1 < _STEPS)
        def _prefetch():
            @pl.when(c >= 1)
            def _():
                wait_store(c - 1, nb)
            fire_gathers(c + 1, nb)

        wait_gathers(b)
        fire_store(c, b)
        return carry

    lax.fori_loop(0, _STEPS, step, 0)
    wait_store(_STEPS - 2, 0)
    wait_store(_STEPS - 1, 1)


def kernel(sentence, word_embed):
    idx = sentence.reshape(_N // _IDX_ROW, _IDX_ROW).astype(jnp.int32)
    packed = _depad_kernel(word_embed.reshape(_SLABS, 8, EMBED_DIM))
    table = packed.reshape(VOCAB, EMBED_DIM)
    out = _gather_kernel(table, idx)
    return out[:, :EMBED_DIM].reshape(BATCH, SEQ, EMBED_DIM)


# speedup vs baseline: 1.1404x; 1.1404x over previous
import functools

import jax
import jax.numpy as jnp
from jax import lax
from jax.experimental import pallas as pl
from jax.experimental.pallas import tpu as pltpu
from jax.experimental.pallas import tpu_sc as plsc

VOCAB = 1000000
EMBED_DIM = 64
BATCH = 4096
SEQ = 200

_N = BATCH * SEQ
_NC = 2
_NS = 16
_NW = _NC * _NS
_PER_W = _N // _NW
_IDX_ROW = 128
_K = 5
_CHUNK = _K * _IDX_ROW
_STEPS = _PER_W // _CHUNK
_PAD = 2 * EMBED_DIM

assert _PER_W % _CHUNK == 0 and _STEPS % 2 == 0


@functools.partial(
    pl.kernel,
    mesh=plsc.VectorSubcoreMesh(core_axis_name="c", subcore_axis_name="s"),
    compiler_params=pltpu.CompilerParams(use_tc_tiling_on_sc=False),
    out_type=jax.ShapeDtypeStruct((_N, _PAD), jnp.float32),
    scratch_types=[
        pltpu.VMEM((2, _K, _IDX_ROW), jnp.int32),
        pltpu.VMEM((2, _CHUNK, EMBED_DIM), jnp.float32),
        pltpu.SemaphoreType.DMA,
        pltpu.SemaphoreType.DMA,
    ],
)
def _gather_kernel(table_hbm, idx_hbm, out_hbm, idx_v, rows_v, gsem, ssem):
    wid = lax.axis_index("s") * _NC + lax.axis_index("c")
    row_base = wid * (_PER_W // _IDX_ROW)
    base = wid * _PER_W

    def fire_gathers(c, b):
        pltpu.sync_copy(idx_hbm.at[pl.ds(row_base + c * _K, _K)], idx_v.at[b])
        for j in range(_K):
            pltpu.async_copy(
                table_hbm.at[idx_v.at[b].at[j]],
                rows_v.at[b].at[pl.ds(j * _IDX_ROW, _IDX_ROW)],
                gsem,
            )

    def wait_gathers(b):
        for j in range(_K):
            pltpu.make_async_copy(
                table_hbm.at[idx_v.at[b].at[j]],
                rows_v.at[b].at[pl.ds(j * _IDX_ROW, _IDX_ROW)],
                gsem,
            ).wait()

    def fire_store(c, b):
        pltpu.async_copy(
            rows_v.at[b],
            out_hbm.at[pl.ds(base + c * _CHUNK, _CHUNK), pl.ds(0, EMBED_DIM)],
            ssem,
        )

    def wait_store(c, b):
        pltpu.make_async_copy(
            rows_v.at[b],
            out_hbm.at[pl.ds(base + c * _CHUNK, _CHUNK), pl.ds(0, EMBED_DIM)],
            ssem,
        ).wait()

    fire_gathers(0, 0)

    def step(c, carry):
        b = lax.rem(c, 2)
        nb = 1 - b

        @pl.when(c + 1 < _STEPS)
        def _prefetch():
            @pl.when(c >= 1)
            def _():
                wait_store(c - 1, nb)
            fire_gathers(c + 1, nb)

        wait_gathers(b)
        fire_store(c, b)
        return carry

    lax.fori_loop(0, _STEPS, step, 0)
    wait_store(_STEPS - 2, 0)
    wait_store(_STEPS - 1, 1)


def kernel(sentence, word_embed):
    idx = sentence.reshape(_N // _IDX_ROW, _IDX_ROW).astype(jnp.int32)
    out = _gather_kernel(word_embed, idx)
    return out[:, :EMBED_DIM].reshape(BATCH, SEQ, EMBED_DIM)
